# baseline (speedup 1.0000x reference)
import jax
import jax.numpy as jnp
from jax import lax
from jax.experimental import pallas as pl
from jax.experimental.pallas import tpu as pltpu

APP = 128
NKPT = 51
FEAT = 184
TOK = 64
BSUB = 8
RN = 2048


def _proj_body(emb_ref, vmb_ref, kpts_ref, w_ref, b_ref, out_ref):
    w = w_ref[...]
    acc = lax.dot_general(
        w[0:APP, :], emb_ref[...],
        (((0,), (2,)), ((), ())),
        preferred_element_type=jnp.float32)
    acc += lax.dot_general(
        w[APP:APP + 5, :], vmb_ref[1:6],
        (((0,), (0,)), ((), ())),
        preferred_element_type=jnp.float32)
    acc += lax.dot_general(
        w[APP + 5:FEAT, :], kpts_ref[...],
        (((0,), (0,)), ((), ())),
        preferred_element_type=jnp.float32)
    acc += b_ref[...].reshape(TOK, 1, 1)
    out_ref[...] = jnp.where(vmb_ref[0:1] != 0, acc, 0.0)


def kernel(embeddings, visibility_scores, bbox_ltwh, keypoints_xyc, feats_masks, W, b):
    Bm, Nm = feats_masks.shape
    vmb = jnp.concatenate([
        feats_masks[None].astype(jnp.float32),
        visibility_scores.reshape(1, Bm, Nm),
        jnp.transpose(bbox_ltwh, (2, 0, 1)),
    ], axis=0)
    kpts_t = jnp.transpose(keypoints_xyc, (2, 3, 0, 1)).reshape(NKPT, Bm, Nm)
    b_col = b.reshape(TOK, 1)

    grid = (Bm // BSUB, Nm // RN)
    out_t = pl.pallas_call(
        _proj_body,
        grid=grid,
        in_specs=[
            pl.BlockSpec((BSUB, RN, APP), lambda i, j: (i, j, 0)),
            pl.BlockSpec((6, BSUB, RN), lambda i, j: (0, i, j)),
            pl.BlockSpec((NKPT, BSUB, RN), lambda i, j: (0, i, j)),
            pl.BlockSpec((FEAT, TOK), lambda i, j: (0, 0)),
            pl.BlockSpec((TOK, 1), lambda i, j: (0, 0)),
        ],
        out_specs=pl.BlockSpec((TOK, BSUB, RN), lambda i, j: (0, i, j)),
        out_shape=jax.ShapeDtypeStruct((TOK, Bm, Nm), jnp.float32),
        compiler_params=pltpu.CompilerParams(
            dimension_semantics=("parallel", "arbitrary"),
        ),
    )(embeddings, vmb, kpts_t, W, b_col)
    return jnp.transpose(out_t, (1, 2, 0))


# speedup vs baseline: 2.4232x; 2.4232x over previous
import jax
import jax.numpy as jnp
from jax import lax
from jax.experimental import pallas as pl
from jax.experimental.pallas import tpu as pltpu

APP = 128
NKPT = 51
FEAT = 184
TOK = 64
BSUB = 8
RN = 2048


def _proj_body(emb_ref, vm_ref, bbox_ref, kpts_ref, w_ref, b_ref, out_ref):
    w = w_ref[...]
    bias = b_ref[...]
    for i in range(BSUB):
        acc = lax.dot_general(
            w[0:APP, :], emb_ref[i],
            (((0,), (1,)), ((), ())),
            preferred_element_type=jnp.float32)
        st_t = jnp.concatenate(
            [vm_ref[1, i:i + 1, :], bbox_ref[i], kpts_ref[:, i, :]],
            axis=0)
        acc += lax.dot_general(
            w[APP:FEAT, :], st_t,
            (((0,), (0,)), ((), ())),
            preferred_element_type=jnp.float32)
        acc += bias
        out_ref[i] = jnp.where(vm_ref[0, i:i + 1, :] != 0, acc, 0.0)


def kernel(embeddings, visibility_scores, bbox_ltwh, keypoints_xyc, feats_masks, W, b):
    Bm, Nm = feats_masks.shape
    mask_f = feats_masks.astype(jnp.float32)
    vm = jnp.stack([mask_f, visibility_scores.reshape(Bm, Nm)])
    bbox_t = jnp.transpose(bbox_ltwh, (0, 2, 1))
    kpts_t = jnp.transpose(keypoints_xyc, (2, 3, 0, 1)).reshape(NKPT, Bm, Nm)
    b_col = b.reshape(TOK, 1)

    grid = (Bm // BSUB, Nm // RN)
    out_t = pl.pallas_call(
        _proj_body,
        grid=grid,
        in_specs=[
            pl.BlockSpec((BSUB, RN, APP), lambda i, j: (i, j, 0)),
            pl.BlockSpec((2, BSUB, RN), lambda i, j: (0, i, j)),
            pl.BlockSpec((BSUB, 4, RN), lambda i, j: (i, 0, j)),
            pl.BlockSpec((NKPT, BSUB, RN), lambda i, j: (0, i, j)),
            pl.BlockSpec((FEAT, TOK), lambda i, j: (0, 0)),
            pl.BlockSpec((TOK, 1), lambda i, j: (0, 0)),
        ],
        out_specs=pl.BlockSpec((BSUB, TOK, RN), lambda i, j: (i, 0, j)),
        out_shape=jax.ShapeDtypeStruct((Bm, TOK, Nm), jnp.float32),
        compiler_params=pltpu.CompilerParams(
            dimension_semantics=("parallel", "arbitrary"),
        ),
    )(embeddings, vm, bbox_t, kpts_t, W, b_col)
    return jnp.transpose(out_t, (0, 2, 1))
